# Initial kernel scaffold; baseline (speedup 1.0000x reference)
import jax
import jax.numpy as jnp
from jax.experimental import pallas as pl


def kernel(x, edge_index, W1, W2):
    raise NotImplementedError("write your pallas kernel here")


# speedup vs baseline: 8.3306x; 8.3306x over previous
import functools

import jax
import jax.numpy as jnp
from jax import lax
from jax.experimental import pallas as pl
from jax.experimental.pallas import tpu as pltpu
from jax.experimental.pallas import tpu_sc as plsc

NC = 2
NS = 16
NW = NC * NS
CHUNK = 128
LANES = 16


def _make_spmm(n, h, npc):
    zbuf_rows = 160
    zrows_tile = -(-(-(-(n + 1) // NS)) // zbuf_rows) * zbuf_rows
    acc_rows = NS * zrows_tile
    mesh = plsc.VectorSubcoreMesh(core_axis_name="c", subcore_axis_name="s")

    @functools.partial(
        pl.kernel,
        mesh=mesh,
        compiler_params=pltpu.CompilerParams(use_tc_tiling_on_sc=False),
        out_type=jax.ShapeDtypeStruct((NC, acc_rows, h), jnp.float32),
        scratch_types=[
            pltpu.VMEM((npc, CHUNK), jnp.int32),
            pltpu.VMEM((npc, CHUNK), jnp.int32),
            pltpu.VMEM((CHUNK, h), jnp.float32),
            pltpu.VMEM((CHUNK, h), jnp.float32),
            pltpu.VMEM((zbuf_rows, h), jnp.float32),
            pltpu.VMEM_SHARED((acc_rows, h), jnp.float32),
            pltpu.SemaphoreType.DMA,
        ],
    )
    def spmm(src_hbm, dst_hbm, tbl_hbm, out_hbm, srcv, dstv, rows0, rows1,
             zbuf, acc, gsem):
        cid = lax.axis_index("c")
        sid = lax.axis_index("s")
        wid = cid * NS + sid

        zero = jnp.zeros((LANES,), jnp.float32)

        def zrow(r, carry):
            for j in range(h // LANES):
                zbuf[r, pl.ds(j * LANES, LANES)] = zero
            return carry

        lax.fori_loop(0, zbuf_rows, zrow, 0)
        for t in range(zrows_tile // zbuf_rows):
            pltpu.sync_copy(
                zbuf, acc.at[pl.ds(sid * zrows_tile + t * zbuf_rows, zbuf_rows)])
        plsc.subcore_barrier()

        pltpu.sync_copy(src_hbm.at[wid], srcv)
        pltpu.sync_copy(dst_hbm.at[wid], dstv)

        def pair(i, carry):
            c0 = pltpu.async_copy(tbl_hbm.at[srcv.at[2 * i]], rows0, gsem)
            c1 = pltpu.async_copy(tbl_hbm.at[srcv.at[2 * i + 1]], rows1, gsem)
            c0.wait()
            pltpu.sync_copy(rows0, acc.at[dstv.at[2 * i]], add=True)
            c1.wait()
            pltpu.sync_copy(rows1, acc.at[dstv.at[2 * i + 1]], add=True)
            return carry

        lax.fori_loop(0, npc // 2, pair, 0)
        plsc.subcore_barrier()

        pltpu.sync_copy(acc.at[pl.ds(sid * zrows_tile, zrows_tile)],
                        out_hbm.at[cid, pl.ds(sid * zrows_tile, zrows_tile)])

    return spmm


def _spmm_partials(edge_index, table):
    n, h = table.shape
    e = edge_index.shape[1]
    per_w = -(-e // (NW * 2 * CHUNK)) * (2 * CHUNK)
    e_pad = per_w * NW
    src = jnp.concatenate(
        [edge_index[0], jnp.zeros((e_pad - e,), jnp.int32)]).reshape(
            NW, per_w // CHUNK, CHUNK)
    dst = jnp.concatenate(
        [edge_index[1], jnp.full((e_pad - e,), n, jnp.int32)]).reshape(
            NW, per_w // CHUNK, CHUNK)
    return _make_spmm(n, h, per_w // CHUNK)(src, dst, table)


def _mm1_body(x_ref, w_ref, o_ref):
    o_ref[...] = jnp.dot(x_ref[...], w_ref[...],
                         preferred_element_type=jnp.float32,
                         precision=lax.Precision.HIGHEST)


def _mm1(x, w, bn):
    n, d = x.shape
    _, h = w.shape
    return pl.pallas_call(
        _mm1_body,
        grid=(n // bn,),
        in_specs=[pl.BlockSpec((bn, d), lambda i: (i, 0)),
                  pl.BlockSpec((d, h), lambda i: (0, 0))],
        out_specs=pl.BlockSpec((bn, h), lambda i: (i, 0)),
        out_shape=jax.ShapeDtypeStruct((n, h), jnp.float32),
    )(x, w)


def _relu_mm_body(p0_ref, p1_ref, w_ref, o_ref):
    hblk = jnp.maximum(p0_ref[0] + p1_ref[0], 0.0)
    o_ref[...] = jnp.dot(hblk, w_ref[...],
                         preferred_element_type=jnp.float32,
                         precision=lax.Precision.HIGHEST)


def _relu_mm(p, w, n, bn):
    _, _, h1 = p.shape
    _, h2 = w.shape
    return pl.pallas_call(
        _relu_mm_body,
        grid=(n // bn,),
        in_specs=[pl.BlockSpec((1, bn, h1), lambda i: (0, i, 0)),
                  pl.BlockSpec((1, bn, h1), lambda i: (1, i, 0)),
                  pl.BlockSpec((h1, h2), lambda i: (0, 0))],
        out_specs=pl.BlockSpec((bn, h2), lambda i: (i, 0)),
        out_shape=jax.ShapeDtypeStruct((n, h2), jnp.float32),
    )(p, p, w)


def _decoder_body(pi0_ref, pi1_ref, pj0_ref, pj1_ref, z_ref, r_ref):
    zi = pi0_ref[0] + pi1_ref[0]
    zj = pj0_ref[0] + pj1_ref[0]
    z_ref[...] = zi
    r_ref[...] = lax.dot_general(
        zi, zj, dimension_numbers=(((1,), (1,)), ((), ())),
        preferred_element_type=jnp.float32,
        precision=lax.Precision.HIGHEST)


def _decoder(p, n, bn):
    _, _, h = p.shape
    return pl.pallas_call(
        _decoder_body,
        grid=(n // bn,),
        in_specs=[pl.BlockSpec((1, bn, h), lambda i: (0, i, 0)),
                  pl.BlockSpec((1, bn, h), lambda i: (1, i, 0)),
                  pl.BlockSpec((1, n, h), lambda i: (0, 0, 0)),
                  pl.BlockSpec((1, n, h), lambda i: (1, 0, 0))],
        out_specs=[pl.BlockSpec((bn, h), lambda i: (i, 0)),
                   pl.BlockSpec((bn, n), lambda i: (i, 0))],
        out_shape=[jax.ShapeDtypeStruct((n, h), jnp.float32),
                   jax.ShapeDtypeStruct((n, n), jnp.float32)],
    )(p, p, p, p)


def kernel(x, edge_index, W1, W2):
    n = x.shape[0]
    h = _mm1(x, W1, 1000)
    p1 = _spmm_partials(edge_index, h)
    h2 = _relu_mm(p1, W2, n, 1000)
    p2 = _spmm_partials(edge_index, h2)
    z, recon = _decoder(p2, n, 400)
    return (z, jnp.reshape(recon, (-1,)))


# speedup vs baseline: 10.2152x; 1.2262x over previous
import functools

import jax
import jax.numpy as jnp
from jax import lax
from jax.experimental import pallas as pl
from jax.experimental.pallas import tpu as pltpu
from jax.experimental.pallas import tpu_sc as plsc

NC = 2
NS = 16
NW = NC * NS
CHUNK = 128
LANES = 16


def _make_spmm(n, h, npc):
    zbuf_rows = 160
    zrows_tile = -(-(-(-(n + 1) // NS)) // zbuf_rows) * zbuf_rows
    acc_rows = NS * zrows_tile
    mesh = plsc.VectorSubcoreMesh(core_axis_name="c", subcore_axis_name="s")

    @functools.partial(
        pl.kernel,
        mesh=mesh,
        compiler_params=pltpu.CompilerParams(use_tc_tiling_on_sc=False),
        out_type=jax.ShapeDtypeStruct((NC, acc_rows, h), jnp.float32),
        scratch_types=[
            pltpu.VMEM((npc, CHUNK), jnp.int32),
            pltpu.VMEM((npc, CHUNK), jnp.int32),
            pltpu.VMEM((CHUNK, h), jnp.float32),
            pltpu.VMEM((CHUNK, h), jnp.float32),
            pltpu.VMEM((zbuf_rows, h), jnp.float32),
            pltpu.VMEM_SHARED((acc_rows, h), jnp.float32),
            pltpu.SemaphoreType.DMA,
        ],
    )
    def spmm(src_hbm, dst_hbm, tbl_hbm, out_hbm, srcv, dstv, rows0, rows1,
             zbuf, acc, gsem):
        cid = lax.axis_index("c")
        sid = lax.axis_index("s")
        wid = cid * NS + sid

        zero = jnp.zeros((LANES,), jnp.float32)

        def zrow(r, carry):
            for j in range(h // LANES):
                zbuf[r, pl.ds(j * LANES, LANES)] = zero
            return carry

        lax.fori_loop(0, zbuf_rows, zrow, 0)
        for t in range(zrows_tile // zbuf_rows):
            pltpu.sync_copy(
                zbuf, acc.at[pl.ds(sid * zrows_tile + t * zbuf_rows, zbuf_rows)])
        plsc.subcore_barrier()

        pltpu.sync_copy(src_hbm.at[wid], srcv)
        pltpu.sync_copy(dst_hbm.at[wid], dstv)

        def pair(i, carry):
            c0 = pltpu.async_copy(tbl_hbm.at[srcv.at[2 * i]], rows0, gsem)
            c1 = pltpu.async_copy(tbl_hbm.at[srcv.at[2 * i + 1]], rows1, gsem)
            c0.wait()
            pltpu.sync_copy(rows0, acc.at[dstv.at[2 * i]], add=True)
            c1.wait()
            pltpu.sync_copy(rows1, acc.at[dstv.at[2 * i + 1]], add=True)
            return carry

        lax.fori_loop(0, npc // 2, pair, 0)
        plsc.subcore_barrier()

        pltpu.sync_copy(acc.at[pl.ds(sid * zrows_tile, zrows_tile)],
                        out_hbm.at[cid, pl.ds(sid * zrows_tile, zrows_tile)])

    return spmm


def _spmm_partials(edge_index, table):
    n, h = table.shape
    e = edge_index.shape[1]
    per_w = -(-e // (NW * 2 * CHUNK)) * (2 * CHUNK)
    e_pad = per_w * NW
    src = jnp.concatenate(
        [edge_index[0], jnp.zeros((e_pad - e,), jnp.int32)]).reshape(
            NW, per_w // CHUNK, CHUNK)
    dst = jnp.concatenate(
        [edge_index[1], jnp.full((e_pad - e,), n, jnp.int32)]).reshape(
            NW, per_w // CHUNK, CHUNK)
    return _make_spmm(n, h, per_w // CHUNK)(src, dst, table)


def _mm1_body(x_ref, w_ref, o_ref):
    o_ref[...] = jnp.dot(x_ref[...], w_ref[...],
                         preferred_element_type=jnp.float32,
                         precision=lax.Precision.HIGHEST)


def _mm1(x, w, bn):
    n, d = x.shape
    _, h = w.shape
    return pl.pallas_call(
        _mm1_body,
        grid=(n // bn,),
        in_specs=[pl.BlockSpec((bn, d), lambda i: (i, 0)),
                  pl.BlockSpec((d, h), lambda i: (0, 0))],
        out_specs=pl.BlockSpec((bn, h), lambda i: (i, 0)),
        out_shape=jax.ShapeDtypeStruct((n, h), jnp.float32),
    )(x, w)


def _relu_mm_body(p0_ref, p1_ref, w_ref, o_ref):
    hblk = jnp.maximum(p0_ref[0] + p1_ref[0], 0.0)
    o_ref[...] = jnp.dot(hblk, w_ref[...],
                         preferred_element_type=jnp.float32,
                         precision=lax.Precision.HIGHEST)


def _relu_mm(p, w, n, bn):
    _, _, h1 = p.shape
    _, h2 = w.shape
    return pl.pallas_call(
        _relu_mm_body,
        grid=(n // bn,),
        in_specs=[pl.BlockSpec((1, bn, h1), lambda i: (0, i, 0)),
                  pl.BlockSpec((1, bn, h1), lambda i: (1, i, 0)),
                  pl.BlockSpec((h1, h2), lambda i: (0, 0))],
        out_specs=pl.BlockSpec((bn, h2), lambda i: (i, 0)),
        out_shape=jax.ShapeDtypeStruct((n, h2), jnp.float32),
    )(p, p, w)


def _decoder_body(pi0_ref, pi1_ref, pj0_ref, pj1_ref, z_ref, r_ref):
    zi = pi0_ref[0] + pi1_ref[0]
    zj = pj0_ref[0] + pj1_ref[0]
    z_ref[...] = zi
    r_ref[...] = lax.dot_general(
        zi, zj, dimension_numbers=(((1,), (1,)), ((), ())),
        preferred_element_type=jnp.float32)


def _decoder(p, n, bn):
    _, _, h = p.shape
    return pl.pallas_call(
        _decoder_body,
        grid=(n // bn,),
        in_specs=[pl.BlockSpec((1, bn, h), lambda i: (0, i, 0)),
                  pl.BlockSpec((1, bn, h), lambda i: (1, i, 0)),
                  pl.BlockSpec((1, n, h), lambda i: (0, 0, 0)),
                  pl.BlockSpec((1, n, h), lambda i: (1, 0, 0))],
        out_specs=[pl.BlockSpec((bn, h), lambda i: (i, 0)),
                   pl.BlockSpec((bn, n), lambda i: (i, 0))],
        out_shape=[jax.ShapeDtypeStruct((n, h), jnp.float32),
                   jax.ShapeDtypeStruct((n, n), jnp.float32)],
    )(p, p, p, p)


def kernel(x, edge_index, W1, W2):
    n = x.shape[0]
    h = _mm1(x, W1, 1000)
    p1 = _spmm_partials(edge_index, h)
    h2 = _relu_mm(p1, W2, n, 1000)
    p2 = _spmm_partials(edge_index, h2)
    z, recon = _decoder(p2, n, 400)
    return (z, jnp.reshape(recon, (-1,)))
